# baseline (speedup 1.0000x reference)
import functools

import jax
import jax.numpy as jnp
from jax import lax
from jax.experimental import pallas as pl
from jax.experimental.pallas import tpu as pltpu
from jax.experimental.pallas import tpu_sc as plsc

_B, _T, _D = 16, 576, 256
_N, _K = 4, 1024
_Dn = _D // _N
_BT = _B * _T
_TB = 1152
_GRID = _BT // _TB
_IC = 128
_NI = _TB // _IC
_NC, _NS = 2, 16

_PREC = lax.Precision.DEFAULT


def _tc_body(z_ref, emb_ref, amin_ref, gidx_ref):
    z = z_ref[...]
    for n in range(_N):
        zn = z[:, n * _Dn:(n + 1) * _Dn]
        en = emb_ref[n]
        mm = lax.dot_general(zn, en, (((1,), (0,)), ((), ())),
                             precision=_PREC,
                             preferred_element_type=jnp.float32)
        znorm = jnp.sum(zn * zn, axis=1, keepdims=True)
        enorm = jnp.sum(en * en, axis=0, keepdims=True)
        dist = znorm - 2.0 * mm + enorm
        amin = jnp.argmin(dist, axis=1).astype(jnp.int32)
        amin_ref[0, n, :] = amin
        gidx_ref[0, n, :] = amin + n * _K


def _tc_stage(zflat, embed):
    return pl.pallas_call(
        _tc_body,
        grid=(_GRID,),
        in_specs=[
            pl.BlockSpec((_TB, _D), lambda i: (i, 0)),
            pl.BlockSpec((_N, _Dn, _K), lambda i: (0, 0, 0)),
        ],
        out_specs=[
            pl.BlockSpec((1, _N, _TB), lambda i: (i, 0, 0)),
            pl.BlockSpec((1, _N, _TB), lambda i: (i, 0, 0)),
        ],
        out_shape=[
            jax.ShapeDtypeStruct((_GRID, _N, _TB), jnp.int32),
            jax.ShapeDtypeStruct((_GRID, _N, _TB), jnp.int32),
        ],
    )(zflat, embed)


@functools.cache
def _sc_gather_kernel():
    mesh = plsc.VectorSubcoreMesh(core_axis_name="c", subcore_axis_name="s")

    @functools.partial(
        pl.kernel,
        out_type=(
            jax.ShapeDtypeStruct((_N, _BT, _Dn), jnp.float32),
            jax.ShapeDtypeStruct((_GRID, _N, _K), jnp.float32),
        ),
        mesh=mesh,
        scratch_types=[
            pltpu.VMEM((_TB,), jnp.int32),
            pltpu.VMEM((_TB, _Dn), jnp.float32),
            pltpu.VMEM((_K,), jnp.float32),
            pltpu.SemaphoreType.DMA,
        ],
        compiler_params=pltpu.CompilerParams(
            use_tc_tiling_on_sc=False, needs_layout_passes=False),
    )
    def _sc_gather(gidx_hbm, table_hbm, out_hbm, hist_hbm,
                   idx_v, rows_v, acc_v, sem):
        c = lax.axis_index("c")
        s = lax.axis_index("s")
        w = s * _NC + c
        g = w // _N
        n = w % _N
        pltpu.sync_copy(gidx_hbm.at[pl.ds(w * _TB, _TB)], idx_v)
        copies = []
        for j in range(_NI):
            copies.append(pltpu.async_copy(
                table_hbm.at[idx_v.at[pl.ds(j * _IC, _IC)]],
                rows_v.at[pl.ds(j * _IC, _IC)], sem))
        for j in range(_K // 16):
            acc_v[pl.ds(j * 16, 16)] = jnp.zeros((16,), jnp.float32)
        base = jnp.full((16,), n * _K, jnp.int32)
        ones = jnp.ones((16,), jnp.float32)
        for j in range(_TB // 16):
            iv = idx_v[pl.ds(j * 16, 16)] - base
            plsc.addupdate_scatter(acc_v, [iv], ones)
        pltpu.sync_copy(acc_v, hist_hbm.at[g, n])
        for cp in copies:
            cp.wait()
        pltpu.sync_copy(rows_v, out_hbm.at[n, pl.ds(g * _TB, _TB)])

    return _sc_gather


def _tc_final_body(z_ref, q_ref, hist_ref, diff_ref, ppl_ref):
    i = pl.program_id(0)

    @pl.when(i == 0)
    def _init():
        diff_ref[...] = jnp.zeros_like(diff_ref)
        ppl_ref[...] = jnp.zeros_like(ppl_ref)

    z = z_ref[...]
    acc = jnp.zeros((1, 1), jnp.float32)
    for n in range(_N):
        r = q_ref[n] - z[:, n * _Dn:(n + 1) * _Dn]
        acc = acc + jnp.sum(r * r).reshape(1, 1)
    diff_ref[...] = diff_ref[...] + acc

    @pl.when(i == _GRID - 1)
    def _finish():
        diff_ref[...] = diff_ref[...] * (1.0 / (_B * _T * _D))
        counts = jnp.sum(hist_ref[...], axis=0)
        probs = counts * (1.0 / _BT)
        ent = -jnp.sum(probs * jnp.log(probs + 1e-10), axis=-1)
        ppl_ref[...] = jnp.mean(jnp.exp(ent)).reshape(1, 1)


def _tc_final(zflat, quant, hist):
    return pl.pallas_call(
        _tc_final_body,
        grid=(_GRID,),
        in_specs=[
            pl.BlockSpec((_TB, _D), lambda i: (i, 0)),
            pl.BlockSpec((_N, _TB, _Dn), lambda i: (0, i, 0)),
            pl.BlockSpec((_GRID, _N, _K), lambda i: (0, 0, 0)),
        ],
        out_specs=[
            pl.BlockSpec((1, 1), lambda i: (0, 0)),
            pl.BlockSpec((1, 1), lambda i: (0, 0)),
        ],
        out_shape=[
            jax.ShapeDtypeStruct((1, 1), jnp.float32),
            jax.ShapeDtypeStruct((1, 1), jnp.float32),
        ],
    )(zflat, quant, hist)


def kernel(z_e, embed):
    zflat = z_e.reshape(_BT, _D)
    amin3, gidx3 = _tc_stage(zflat, embed)
    codes = jnp.transpose(embed, (0, 2, 1)).reshape(_N * _K, _Dn)
    quant, hist = _sc_gather_kernel()(gidx3.reshape(_GRID * _N * _TB), codes)
    diff, ppl = _tc_final(zflat, quant, hist)
    z_q = jnp.transpose(quant.reshape(_N, _B, _T, _Dn),
                        (1, 2, 0, 3)).reshape(_B, _T, _D)
    argmin = jnp.transpose(amin3, (1, 0, 2)).reshape(_N, _BT)
    return z_q, diff.reshape(()), ppl.reshape(()), argmin


# speedup vs baseline: 1.2031x; 1.2031x over previous
import functools

import jax
import jax.numpy as jnp
from jax import lax
from jax.experimental import pallas as pl
from jax.experimental.pallas import tpu as pltpu
from jax.experimental.pallas import tpu_sc as plsc

_B, _T, _D = 16, 576, 256
_N, _K = 4, 1024
_Dn = _D // _N
_BT = _B * _T
_TB = 1152
_GRID = _BT // _TB
_IC = 128
_NI = _TB // _IC
_NC, _NS = 2, 16

_PREC = lax.Precision.DEFAULT


def _tc_body(z_ref, emb_ref, amin_ref, gidx_ref):
    z = z_ref[...]
    for n in range(_N):
        zn = z[:, n * _Dn:(n + 1) * _Dn]
        en = emb_ref[n]
        mm2 = lax.dot_general(zn, -2.0 * en, (((1,), (0,)), ((), ())),
                              precision=_PREC,
                              preferred_element_type=jnp.float32)
        znorm = jnp.sum(zn * zn, axis=1, keepdims=True)
        enorm = jnp.sum(en * en, axis=0, keepdims=True)
        dist = (znorm + mm2) + enorm
        amin = jnp.argmin(dist, axis=1).astype(jnp.int32)
        amin_ref[n] = amin.reshape(_NI, _IC)
        gidx_ref[n] = (amin + n * _K).reshape(_NI, _IC)


def _tc_stage(zflat, embed):
    return pl.pallas_call(
        _tc_body,
        grid=(_GRID,),
        in_specs=[
            pl.BlockSpec((_TB, _D), lambda i: (i, 0)),
            pl.BlockSpec((_N, _Dn, _K), lambda i: (0, 0, 0)),
        ],
        out_specs=[
            pl.BlockSpec((_N, _NI, _IC), lambda i: (i, 0, 0)),
            pl.BlockSpec((_N, _NI, _IC), lambda i: (i, 0, 0)),
        ],
        out_shape=[
            jax.ShapeDtypeStruct((_GRID * _N, _NI, _IC), jnp.int32),
            jax.ShapeDtypeStruct((_GRID * _N, _NI, _IC), jnp.int32),
        ],
    )(zflat, embed)


@functools.cache
def _sc_gather_kernel():
    mesh = plsc.VectorSubcoreMesh(core_axis_name="c", subcore_axis_name="s")

    @functools.partial(
        pl.kernel,
        out_type=(
            jax.ShapeDtypeStruct((_BT, _N, _Dn), jnp.float32),
            jax.ShapeDtypeStruct((_GRID, _N, _K), jnp.float32),
        ),
        mesh=mesh,
        scratch_types=[
            pltpu.VMEM((_TB,), jnp.int32),
            pltpu.VMEM((_TB, _Dn), jnp.float32),
            pltpu.VMEM((_K,), jnp.float32),
            pltpu.SemaphoreType.DMA,
        ],
        compiler_params=pltpu.CompilerParams(
            use_tc_tiling_on_sc=False, needs_layout_passes=False),
    )
    def _sc_gather(gidx_hbm, table_hbm, out_hbm, hist_hbm,
                   idx_v, rows_v, acc_v, sem):
        c = lax.axis_index("c")
        s = lax.axis_index("s")
        w = s * _NC + c
        g = w // _N
        n = w % _N
        pltpu.sync_copy(gidx_hbm.at[pl.ds(w * _TB, _TB)], idx_v)
        copies = []
        for j in range(_NI):
            copies.append(pltpu.async_copy(
                table_hbm.at[idx_v.at[pl.ds(j * _IC, _IC)]],
                rows_v.at[pl.ds(j * _IC, _IC)], sem))
        for j in range(_K // 16):
            acc_v[pl.ds(j * 16, 16)] = jnp.zeros((16,), jnp.float32)
        base = jnp.full((16,), n * _K, jnp.int32)
        ones = jnp.ones((16,), jnp.float32)
        for j in range(_TB // 16):
            iv = idx_v[pl.ds(j * 16, 16)] - base
            plsc.addupdate_scatter(acc_v, [iv], ones)
        pltpu.sync_copy(acc_v, hist_hbm.at[g, n])
        for cp in copies:
            cp.wait()
        pltpu.sync_copy(rows_v, out_hbm.at[pl.ds(g * _TB, _TB), n])

    return _sc_gather


def _tc_final_body(z_ref, q_ref, hist_ref, diff_ref, ppl_ref):
    i = pl.program_id(0)

    @pl.when(i == 0)
    def _init():
        diff_ref[...] = jnp.zeros_like(diff_ref)
        ppl_ref[...] = jnp.zeros_like(ppl_ref)

    z = z_ref[...]
    q = q_ref[...]
    acc = jnp.zeros((1, 1), jnp.float32)
    for n in range(_N):
        r = q[:, n, :] - z[:, n * _Dn:(n + 1) * _Dn]
        acc = acc + jnp.sum(r * r).reshape(1, 1)
    diff_ref[...] = diff_ref[...] + acc

    @pl.when(i == _GRID - 1)
    def _finish():
        diff_ref[...] = diff_ref[...] * (1.0 / (_B * _T * _D))
        counts = jnp.sum(hist_ref[...], axis=0)
        probs = counts * (1.0 / _BT)
        ent = -jnp.sum(probs * jnp.log(probs + 1e-10), axis=-1)
        ppl_ref[...] = jnp.mean(jnp.exp(ent)).reshape(1, 1)


def _tc_final(zflat, quant, hist):
    return pl.pallas_call(
        _tc_final_body,
        grid=(_GRID,),
        in_specs=[
            pl.BlockSpec((_TB, _D), lambda i: (i, 0)),
            pl.BlockSpec((_TB, _N, _Dn), lambda i: (i, 0, 0)),
            pl.BlockSpec((_GRID, _N, _K), lambda i: (0, 0, 0)),
        ],
        out_specs=[
            pl.BlockSpec((1, 1), lambda i: (0, 0)),
            pl.BlockSpec((1, 1), lambda i: (0, 0)),
        ],
        out_shape=[
            jax.ShapeDtypeStruct((1, 1), jnp.float32),
            jax.ShapeDtypeStruct((1, 1), jnp.float32),
        ],
    )(zflat, quant, hist)


def kernel(z_e, embed):
    zflat = z_e.reshape(_BT, _D)
    amin3, gidx3 = _tc_stage(zflat, embed)
    codes = jnp.transpose(embed, (0, 2, 1)).reshape(_N * _K, _Dn)
    quant, hist = _sc_gather_kernel()(gidx3.reshape(_GRID * _N * _TB), codes)
    diff, ppl = _tc_final(zflat, quant, hist)
    z_q = quant.reshape(_B, _T, _D)
    argmin = jnp.transpose(amin3.reshape(_GRID, _N, _TB),
                           (1, 0, 2)).reshape(_N, _BT)
    return z_q, diff.reshape(()), ppl.reshape(()), argmin


# speedup vs baseline: 1.4993x; 1.2462x over previous
import functools

import jax
import jax.numpy as jnp
from jax import lax
from jax.experimental import pallas as pl
from jax.experimental.pallas import tpu as pltpu
from jax.experimental.pallas import tpu_sc as plsc

_B, _T, _D = 16, 576, 256
_N, _K = 4, 1024
_Dn = _D // _N
_BT = _B * _T
_TB = 1152
_GRID = _BT // _TB
_IC = 128
_NI = _TB // _IC
_NC, _NS = 2, 16

_PREC = lax.Precision.DEFAULT


def _tc_body(z_ref, emb_ref, amin_ref, gidx_ref):
    z = z_ref[...]
    for n in range(_N):
        zn = z[:, n * _Dn:(n + 1) * _Dn]
        en = emb_ref[n]
        mm2 = lax.dot_general(zn, -2.0 * en, (((1,), (0,)), ((), ())),
                              precision=_PREC,
                              preferred_element_type=jnp.float32)
        znorm = jnp.sum(zn * zn, axis=1, keepdims=True)
        enorm = jnp.sum(en * en, axis=0, keepdims=True)
        dist = (znorm + mm2) + enorm
        amin = jnp.argmin(dist, axis=1).astype(jnp.int32)
        amin_ref[n] = amin.reshape(_NI, _IC)
        gidx_ref[n] = (amin + n * _K).reshape(_NI, _IC)


def _tc_stage(zflat, embed):
    return pl.pallas_call(
        _tc_body,
        grid=(_GRID,),
        in_specs=[
            pl.BlockSpec((_TB, _D), lambda i: (i, 0)),
            pl.BlockSpec((_N, _Dn, _K), lambda i: (0, 0, 0)),
        ],
        out_specs=[
            pl.BlockSpec((_N, _NI, _IC), lambda i: (i, 0, 0)),
            pl.BlockSpec((_N, _NI, _IC), lambda i: (i, 0, 0)),
        ],
        out_shape=[
            jax.ShapeDtypeStruct((_GRID * _N, _NI, _IC), jnp.int32),
            jax.ShapeDtypeStruct((_GRID * _N, _NI, _IC), jnp.int32),
        ],
    )(zflat, embed)


@functools.cache
def _sc_gather_kernel():
    mesh = plsc.VectorSubcoreMesh(core_axis_name="c", subcore_axis_name="s")

    @functools.partial(
        pl.kernel,
        out_type=(
            jax.ShapeDtypeStruct((_BT, _D), jnp.float32),
            jax.ShapeDtypeStruct((_GRID, _N, _K), jnp.float32),
        ),
        mesh=mesh,
        scratch_types=[
            pltpu.VMEM((_TB,), jnp.int32),
            pltpu.VMEM((_TB, _Dn), jnp.float32),
            pltpu.VMEM((_K,), jnp.float32),
            pltpu.SemaphoreType.DMA,
        ],
        compiler_params=pltpu.CompilerParams(
            use_tc_tiling_on_sc=False, needs_layout_passes=False),
    )
    def _sc_gather(gidx_hbm, table_hbm, out_hbm, hist_hbm,
                   idx_v, rows_v, acc_v, sem):
        c = lax.axis_index("c")
        s = lax.axis_index("s")
        w = s * _NC + c
        g = w // _N
        n = w % _N
        pltpu.sync_copy(gidx_hbm.at[pl.ds(w * _TB, _TB)], idx_v)
        copies = []
        for j in range(_NI):
            copies.append(pltpu.async_copy(
                table_hbm.at[idx_v.at[pl.ds(j * _IC, _IC)]],
                rows_v.at[pl.ds(j * _IC, _IC)], sem))
        for j in range(_K // 16):
            acc_v[pl.ds(j * 16, 16)] = jnp.zeros((16,), jnp.float32)
        base = jnp.full((16,), n * _K, jnp.int32)
        ones = jnp.ones((16,), jnp.float32)
        for j in range(_TB // 16):
            iv = idx_v[pl.ds(j * 16, 16)] - base
            plsc.addupdate_scatter(acc_v, [iv], ones)
        pltpu.sync_copy(acc_v, hist_hbm.at[g, n])
        for cp in copies:
            cp.wait()
        pltpu.sync_copy(rows_v,
                        out_hbm.at[pl.ds(g * _TB, _TB), pl.ds(n * _Dn, _Dn)])

    return _sc_gather


def _tc_final_body(z_ref, q_ref, hist_ref, diff_ref, ppl_ref):
    i = pl.program_id(0)

    @pl.when(i == 0)
    def _init():
        diff_ref[...] = jnp.zeros_like(diff_ref)
        ppl_ref[...] = jnp.zeros_like(ppl_ref)

    r = q_ref[...] - z_ref[...]
    acc = jnp.sum(r * r).reshape(1, 1)
    diff_ref[...] = diff_ref[...] + acc

    @pl.when(i == _GRID - 1)
    def _finish():
        diff_ref[...] = diff_ref[...] * (1.0 / (_B * _T * _D))
        counts = jnp.sum(hist_ref[...], axis=0)
        probs = counts * (1.0 / _BT)
        ent = -jnp.sum(probs * jnp.log(probs + 1e-10), axis=-1)
        ppl_ref[...] = jnp.mean(jnp.exp(ent)).reshape(1, 1)


def _tc_final(zflat, quant, hist):
    return pl.pallas_call(
        _tc_final_body,
        grid=(_GRID,),
        in_specs=[
            pl.BlockSpec((_TB, _D), lambda i: (i, 0)),
            pl.BlockSpec((_TB, _D), lambda i: (i, 0)),
            pl.BlockSpec((_GRID, _N, _K), lambda i: (0, 0, 0)),
        ],
        out_specs=[
            pl.BlockSpec((1, 1), lambda i: (0, 0)),
            pl.BlockSpec((1, 1), lambda i: (0, 0)),
        ],
        out_shape=[
            jax.ShapeDtypeStruct((1, 1), jnp.float32),
            jax.ShapeDtypeStruct((1, 1), jnp.float32),
        ],
    )(zflat, quant, hist)


def kernel(z_e, embed):
    zflat = z_e.reshape(_BT, _D)
    amin3, gidx3 = _tc_stage(zflat, embed)
    codes = jnp.transpose(embed, (0, 2, 1)).reshape(_N * _K, _Dn)
    quant, hist = _sc_gather_kernel()(gidx3.reshape(_GRID * _N * _TB), codes)
    diff, ppl = _tc_final(zflat, quant, hist)
    z_q = quant.reshape(_B, _T, _D)
    argmin = jnp.transpose(amin3.reshape(_GRID, _N, _TB),
                           (1, 0, 2)).reshape(_N, _BT)
    return z_q, diff.reshape(()), ppl.reshape(()), argmin
